# baseline (device time: 16847 ns/iter reference)
import jax
import jax.numpy as jnp
from jax import lax
from jax.experimental import pallas as pl
from jax.experimental.pallas import tpu as pltpu

N_DEV = 4
ALIGN = 8
CAP = 544
MAX_UBITS = 7
SUB = 4
LANES = 128


def _barrier(peers, n):
    barrier = pltpu.get_barrier_semaphore()
    for nbr in peers:
        pl.semaphore_signal(
            barrier,
            inc=1,
            device_id=(nbr,),
            device_id_type=pl.DeviceIdType.MESH,
        )
    pl.semaphore_wait(barrier, n)


def _sel(pred_scalar_pairs):
    acc = None
    for p, v in pred_scalar_pairs:
        term = jnp.where(p, v, 0)
        acc = term if acc is None else acc + term
    return acc


def _plan(d_shard):

    def body(d_ref, slot_ref, su_ref, so_ref):
        my = lax.axis_index("i")
        dloc = d_ref[:, :]
        masks = [(dloc == d).astype(jnp.int32) for d in range(N_DEV)]
        c = [jnp.sum(masks[d]) for d in range(N_DEV)]
        cp = [((c[d] + ALIGN - 1) // ALIGN) * ALIGN for d in range(N_DEV)]
        ploff = []
        acc = 0
        for d in range(N_DEV):
            ploff.append(acc)
            acc = acc + cp[d]

        ri = lax.broadcasted_iota(jnp.int32, (SUB, LANES), 0)
        upper = (
            lax.broadcasted_iota(jnp.int32, (LANES, LANES), 0)
            <= lax.broadcasted_iota(jnp.int32, (LANES, LANES), 1)
        ).astype(jnp.float32)
        slot = jnp.zeros((SUB, LANES), jnp.int32)
        for d in range(N_DEV):
            mf = masks[d].astype(jnp.float32)
            incl = jax.lax.dot_general(
                mf, upper, (((1,), (0,)), ((), ())),
                preferred_element_type=jnp.float32,
            )
            rowtot = [jnp.sum(mf[r]) for r in range(SUB)]
            roacc = 0.0
            roff = jnp.zeros((SUB, LANES), jnp.float32)
            for r in range(SUB):
                roff = roff + jnp.where(
                    ri == r, jnp.float32(1), jnp.float32(0)
                ) * roacc
                roacc = roacc + rowtot[r]
            rank = (roff + incl).astype(jnp.int32) - masks[d]
            slot = slot + masks[d] * (ploff[d] + rank)
        slot_ref[:, :] = slot

        for k in range(N_DEV):
            tgt = lax.rem(my + k, N_DEV)
            su_ref[k] = _sel([(tgt == d, cp[d]) for d in range(N_DEV)]) // ALIGN
            so_ref[k] = _sel([(tgt == d, ploff[d]) for d in range(N_DEV)])

    return pl.pallas_call(
        body,
        out_shape=(
            jax.ShapeDtypeStruct((SUB, LANES), jnp.int32),
            jax.ShapeDtypeStruct((N_DEV,), jnp.int32),
            jax.ShapeDtypeStruct((N_DEV,), jnp.int32),
        ),
        in_specs=[pl.BlockSpec(memory_space=pltpu.VMEM)],
        out_specs=(
            pl.BlockSpec(memory_space=pltpu.VMEM),
            pl.BlockSpec(memory_space=pltpu.SMEM),
            pl.BlockSpec(memory_space=pltpu.SMEM),
        ),
    )(d_shard)


def _a2av(x_sp, d_shard, su, so):
    cap, n = x_sp.shape

    def body(x_ref, d_ref, su_ref, so_ref, out_ref, idx_ref,
             dg_ref, xsend, xrecv, dsend, drecv):
        my = lax.axis_index("i")
        peers = tuple(lax.rem(my + k, N_DEV) for k in range(1, N_DEV))
        _barrier(peers, N_DEV - 1)

        dg_ref[my] = d_ref[:, :]
        d_rdmas = []
        for k, tgt in enumerate(peers):
            rdma = pltpu.make_async_remote_copy(
                src_ref=dg_ref.at[my],
                dst_ref=dg_ref.at[my],
                send_sem=dsend.at[k],
                recv_sem=drecv.at[k],
                device_id=(tgt,),
                device_id_type=pl.DeviceIdType.MESH,
            )
            rdma.start()
            d_rdmas.append(rdma)

        for k in range(1, N_DEV):
            tgt = lax.rem(my + k, N_DEV)
            cnt = su_ref[k]
            srcp, dstp = pl.multiple_of(so_ref[k], ALIGN), 0
            for b in reversed(range(MAX_UBITS)):
                sz = ALIGN << b
                bit = lax.shift_right_logical(cnt, b) & 1

                @pl.when(bit == 1)
                def _(srcp=srcp, dstp=dstp, sz=sz, k=k, tgt=tgt):
                    rdma = pltpu.make_async_remote_copy(
                        src_ref=x_ref.at[pl.ds(srcp, sz)],
                        dst_ref=out_ref.at[my].at[pl.ds(dstp, sz)],
                        send_sem=xsend.at[k],
                        recv_sem=xrecv.at[(N_DEV - k) % N_DEV],
                        device_id=(tgt,),
                        device_id_type=pl.DeviceIdType.MESH,
                    )
                    rdma.start()

                srcp = pl.multiple_of(srcp + bit * sz, ALIGN)
                dstp = pl.multiple_of(dstp + bit * sz, ALIGN)

        cnt0 = su_ref[0]
        srcp, dstp = pl.multiple_of(so_ref[0], ALIGN), 0
        for b in reversed(range(MAX_UBITS)):
            sz = ALIGN << b
            bit = lax.shift_right_logical(cnt0, b) & 1

            @pl.when(bit == 1)
            def _(srcp=srcp, dstp=dstp, sz=sz):
                out_ref[my, pl.ds(dstp, sz), :] = x_ref[pl.ds(srcp, sz), :]

            srcp = pl.multiple_of(srcp + bit * sz, ALIGN)
            dstp = pl.multiple_of(dstp + bit * sz, ALIGN)

        for rdma in d_rdmas:
            rdma.wait()

        rcA = [
            jnp.sum((dg_ref[s] == my).astype(jnp.int32))
            for s in range(N_DEV)
        ]
        rincl = []
        acc = 0
        for s in range(N_DEV):
            acc = acc + rcA[s]
            rincl.append(acc)
        rexcl = [rincl[s] - rcA[s] for s in range(N_DEV)]

        li = lax.broadcasted_iota(jnp.int32, (SUB, LANES), 1)
        ri = lax.broadcasted_iota(jnp.int32, (SUB, LANES), 0)
        t = ri * LANES + li
        j_of = jnp.zeros((SUB, LANES), jnp.int32)
        for s in range(N_DEV - 1):
            j_of = j_of + (t >= rincl[s]).astype(jnp.int32)
        base = jnp.zeros((SUB, LANES), jnp.int32)
        for s in range(N_DEV):
            base = base + jnp.where(j_of == s, rexcl[s], 0)
        idx_ref[:, :] = j_of * CAP + (t - base)

        for k in range(1, N_DEV):
            cnt = su_ref[k]
            for b in reversed(range(MAX_UBITS)):
                sz = ALIGN << b
                bit = lax.shift_right_logical(cnt, b) & 1

                @pl.when(bit == 1)
                def _(sz=sz, k=k):
                    dummy = pltpu.make_async_remote_copy(
                        src_ref=x_ref.at[pl.ds(0, sz)],
                        dst_ref=out_ref.at[0].at[pl.ds(0, sz)],
                        send_sem=xsend.at[k],
                        recv_sem=xrecv.at[(N_DEV - k) % N_DEV],
                        device_id=(my,),
                        device_id_type=pl.DeviceIdType.MESH,
                    )
                    dummy.wait_send()

        for j in range(1, N_DEV):
            src = lax.rem(my + j, N_DEV)
            rcj = _sel([(src == s, rcA[s]) for s in range(N_DEV)])
            cnt = (rcj + ALIGN - 1) // ALIGN
            for b in reversed(range(MAX_UBITS)):
                sz = ALIGN << b
                bit = lax.shift_right_logical(cnt, b) & 1

                @pl.when(bit == 1)
                def _(sz=sz, j=j):
                    dummy = pltpu.make_async_remote_copy(
                        src_ref=x_ref.at[pl.ds(0, sz)],
                        dst_ref=out_ref.at[0].at[pl.ds(0, sz)],
                        send_sem=xsend.at[j],
                        recv_sem=xrecv.at[j],
                        device_id=(my,),
                        device_id_type=pl.DeviceIdType.MESH,
                    )
                    dummy.wait_recv()

    return pl.pallas_call(
        body,
        out_shape=(
            jax.ShapeDtypeStruct((N_DEV, cap, n), x_sp.dtype),
            jax.ShapeDtypeStruct((SUB, LANES), jnp.int32),
        ),
        in_specs=[
            pl.BlockSpec(memory_space=pltpu.VMEM),
            pl.BlockSpec(memory_space=pltpu.VMEM),
            pl.BlockSpec(memory_space=pltpu.SMEM),
            pl.BlockSpec(memory_space=pltpu.SMEM),
        ],
        out_specs=(
            pl.BlockSpec(memory_space=pltpu.VMEM),
            pl.BlockSpec(memory_space=pltpu.VMEM),
        ),
        scratch_shapes=[
            pltpu.VMEM((N_DEV, SUB, LANES), jnp.int32),
            pltpu.SemaphoreType.DMA((N_DEV,)),
            pltpu.SemaphoreType.DMA((N_DEV,)),
            pltpu.SemaphoreType.DMA((N_DEV - 1,)),
            pltpu.SemaphoreType.DMA((N_DEV - 1,)),
        ],
        compiler_params=pltpu.CompilerParams(collective_id=0),
    )(x_sp, d_shard, su, so)


def kernel(x, dest):
    m, n = x.shape
    d2 = dest.reshape(SUB, LANES)
    slot, su, so = _plan(d2)

    inv = (
        jnp.zeros((CAP,), jnp.int32)
        .at[slot.reshape(m)]
        .set(jnp.arange(m, dtype=jnp.int32), unique_indices=True)
    )
    x_sp = jnp.take(x, inv, axis=0)

    regions, idx = _a2av(x_sp, d2, su, so)

    return jnp.take(regions.reshape(N_DEV * CAP, n), idx.reshape(m), axis=0)


# device time: 15012 ns/iter; 1.1222x vs baseline; 1.1222x over previous
import jax
import jax.numpy as jnp
from jax import lax
from jax.experimental import pallas as pl
from jax.experimental.pallas import tpu as pltpu

N_DEV = 4
ALIGN = 8
CAP = 544
MAX_UBITS = 7
SUB = 4
LANES = 128


def _barrier(peers, n):
    barrier = pltpu.get_barrier_semaphore()
    for nbr in peers:
        pl.semaphore_signal(
            barrier,
            inc=1,
            device_id=(nbr,),
            device_id_type=pl.DeviceIdType.MESH,
        )
    pl.semaphore_wait(barrier, n)


def _sel(pred_scalar_pairs):
    acc = None
    for p, v in pred_scalar_pairs:
        term = jnp.where(p, v, 0)
        acc = term if acc is None else acc + term
    return acc


def _plan(d_shard):

    def body(d_ref, slot_ref, su_ref, so_ref):
        my = lax.axis_index("i")
        dloc = d_ref[:, :]
        masks = [(dloc == d).astype(jnp.int32) for d in range(N_DEV)]
        c = [jnp.sum(masks[d]) for d in range(N_DEV)]
        cp = [((c[d] + ALIGN - 1) // ALIGN) * ALIGN for d in range(N_DEV)]
        ploff = []
        acc = 0
        for d in range(N_DEV):
            ploff.append(acc)
            acc = acc + cp[d]

        ri = lax.broadcasted_iota(jnp.int32, (SUB, LANES), 0)
        upper = (
            lax.broadcasted_iota(jnp.int32, (LANES, LANES), 0)
            <= lax.broadcasted_iota(jnp.int32, (LANES, LANES), 1)
        ).astype(jnp.float32)
        slot = jnp.zeros((SUB, LANES), jnp.int32)
        for d in range(N_DEV):
            mf = masks[d].astype(jnp.float32)
            incl = jax.lax.dot_general(
                mf, upper, (((1,), (0,)), ((), ())),
                preferred_element_type=jnp.float32,
            )
            rowtot = [jnp.sum(mf[r]) for r in range(SUB)]
            roacc = 0.0
            roff = jnp.zeros((SUB, LANES), jnp.float32)
            for r in range(SUB):
                roff = roff + jnp.where(
                    ri == r, jnp.float32(1), jnp.float32(0)
                ) * roacc
                roacc = roacc + rowtot[r]
            rank = (roff + incl).astype(jnp.int32) - masks[d]
            slot = slot + masks[d] * (ploff[d] + rank)
        slot_ref[:, :] = slot

        for k in range(N_DEV):
            tgt = lax.rem(my + k, N_DEV)
            su_ref[k] = _sel([(tgt == d, cp[d]) for d in range(N_DEV)]) // ALIGN
            so_ref[k] = _sel([(tgt == d, ploff[d]) for d in range(N_DEV)])

    return pl.pallas_call(
        body,
        out_shape=(
            jax.ShapeDtypeStruct((SUB, LANES), jnp.int32),
            jax.ShapeDtypeStruct((N_DEV,), jnp.int32),
            jax.ShapeDtypeStruct((N_DEV,), jnp.int32),
        ),
        in_specs=[pl.BlockSpec(memory_space=pltpu.VMEM)],
        out_specs=(
            pl.BlockSpec(memory_space=pltpu.VMEM),
            pl.BlockSpec(memory_space=pltpu.SMEM),
            pl.BlockSpec(memory_space=pltpu.SMEM),
        ),
    )(d_shard)


def _a2av(x_sp, d_shard, su, so):
    cap, n = x_sp.shape

    def body(x_ref, d_ref, su_ref, so_ref, out_ref, idx_ref,
             dg_ref, xsend, xrecv, dsend, drecv):
        my = lax.axis_index("i")
        peers = tuple(lax.rem(my + k, N_DEV) for k in range(1, N_DEV))
        _barrier(peers, N_DEV - 1)

        dg_ref[my] = d_ref[:, :]
        d_rdmas = []
        for k, tgt in enumerate(peers):
            rdma = pltpu.make_async_remote_copy(
                src_ref=dg_ref.at[my],
                dst_ref=dg_ref.at[my],
                send_sem=dsend.at[k],
                recv_sem=drecv.at[k],
                device_id=(tgt,),
                device_id_type=pl.DeviceIdType.MESH,
            )
            rdma.start()
            d_rdmas.append(rdma)

        for k in range(1, N_DEV):
            tgt = lax.rem(my + k, N_DEV)
            cnt = su_ref[k]
            srcp, dstp = pl.multiple_of(so_ref[k], ALIGN), 0
            for b in reversed(range(MAX_UBITS)):
                sz = ALIGN << b
                bit = lax.shift_right_logical(cnt, b) & 1

                @pl.when(bit == 1)
                def _(srcp=srcp, dstp=dstp, sz=sz, k=k, tgt=tgt):
                    rdma = pltpu.make_async_remote_copy(
                        src_ref=x_ref.at[pl.ds(srcp, sz)],
                        dst_ref=out_ref.at[my].at[pl.ds(dstp, sz)],
                        send_sem=xsend.at[k],
                        recv_sem=xrecv.at[(N_DEV - k) % N_DEV],
                        device_id=(tgt,),
                        device_id_type=pl.DeviceIdType.MESH,
                    )
                    rdma.start()

                srcp = pl.multiple_of(srcp + bit * sz, ALIGN)
                dstp = pl.multiple_of(dstp + bit * sz, ALIGN)

        cnt0 = su_ref[0]
        srcp, dstp = pl.multiple_of(so_ref[0], ALIGN), 0
        for b in reversed(range(MAX_UBITS)):
            sz = ALIGN << b
            bit = lax.shift_right_logical(cnt0, b) & 1

            @pl.when(bit == 1)
            def _(srcp=srcp, dstp=dstp, sz=sz):
                out_ref[my, pl.ds(dstp, sz), :] = x_ref[pl.ds(srcp, sz), :]

            srcp = pl.multiple_of(srcp + bit * sz, ALIGN)
            dstp = pl.multiple_of(dstp + bit * sz, ALIGN)

        for rdma in d_rdmas:
            rdma.wait()

        rcA = [
            jnp.sum((dg_ref[s] == my).astype(jnp.int32))
            for s in range(N_DEV)
        ]
        rincl = []
        acc = 0
        for s in range(N_DEV):
            acc = acc + rcA[s]
            rincl.append(acc)
        rexcl = [rincl[s] - rcA[s] for s in range(N_DEV)]

        li = lax.broadcasted_iota(jnp.int32, (SUB, LANES), 1)
        ri = lax.broadcasted_iota(jnp.int32, (SUB, LANES), 0)
        t = ri * LANES + li
        j_of = jnp.zeros((SUB, LANES), jnp.int32)
        for s in range(N_DEV - 1):
            j_of = j_of + (t >= rincl[s]).astype(jnp.int32)
        base = jnp.zeros((SUB, LANES), jnp.int32)
        for s in range(N_DEV):
            base = base + jnp.where(j_of == s, rexcl[s], 0)
        idx_ref[:, :] = j_of * CAP + (t - base)

        for k in range(1, N_DEV):
            cnt = su_ref[k]
            for b in reversed(range(MAX_UBITS)):
                sz = ALIGN << b
                bit = lax.shift_right_logical(cnt, b) & 1

                @pl.when(bit == 1)
                def _(sz=sz, k=k):
                    dummy = pltpu.make_async_remote_copy(
                        src_ref=x_ref.at[pl.ds(0, sz)],
                        dst_ref=out_ref.at[0].at[pl.ds(0, sz)],
                        send_sem=xsend.at[k],
                        recv_sem=xrecv.at[(N_DEV - k) % N_DEV],
                        device_id=(my,),
                        device_id_type=pl.DeviceIdType.MESH,
                    )
                    dummy.wait_send()

        for j in range(1, N_DEV):
            src = lax.rem(my + j, N_DEV)
            rcj = _sel([(src == s, rcA[s]) for s in range(N_DEV)])
            cnt = (rcj + ALIGN - 1) // ALIGN
            for b in reversed(range(MAX_UBITS)):
                sz = ALIGN << b
                bit = lax.shift_right_logical(cnt, b) & 1

                @pl.when(bit == 1)
                def _(sz=sz, j=j):
                    dummy = pltpu.make_async_remote_copy(
                        src_ref=x_ref.at[pl.ds(0, sz)],
                        dst_ref=out_ref.at[0].at[pl.ds(0, sz)],
                        send_sem=xsend.at[j],
                        recv_sem=xrecv.at[j],
                        device_id=(my,),
                        device_id_type=pl.DeviceIdType.MESH,
                    )
                    dummy.wait_recv()

    return pl.pallas_call(
        body,
        out_shape=(
            jax.ShapeDtypeStruct((N_DEV, cap, n), x_sp.dtype),
            jax.ShapeDtypeStruct((SUB, LANES), jnp.int32),
        ),
        in_specs=[
            pl.BlockSpec(memory_space=pltpu.VMEM),
            pl.BlockSpec(memory_space=pltpu.VMEM),
            pl.BlockSpec(memory_space=pltpu.SMEM),
            pl.BlockSpec(memory_space=pltpu.SMEM),
        ],
        out_specs=(
            pl.BlockSpec(memory_space=pltpu.VMEM),
            pl.BlockSpec(memory_space=pltpu.VMEM),
        ),
        scratch_shapes=[
            pltpu.VMEM((N_DEV, SUB, LANES), jnp.int32),
            pltpu.SemaphoreType.DMA((N_DEV,)),
            pltpu.SemaphoreType.DMA((N_DEV,)),
            pltpu.SemaphoreType.DMA((N_DEV - 1,)),
            pltpu.SemaphoreType.DMA((N_DEV - 1,)),
        ],
        compiler_params=pltpu.CompilerParams(collective_id=0),
    )(x_sp, d_shard, su, so)


def kernel(x, dest):
    m, n = x.shape
    d2 = dest.reshape(SUB, LANES)
    slot, su, so = _plan(d2)

    x_sp = (
        jnp.zeros((CAP, n), x.dtype)
        .at[slot.reshape(m)]
        .set(x, unique_indices=True, indices_are_sorted=False)
    )

    regions, idx = _a2av(x_sp, d2, su, so)

    return jnp.take(regions.reshape(N_DEV * CAP, n), idx.reshape(m), axis=0)
